# baseline (speedup 1.0000x reference)
import functools

import jax
import jax.numpy as jnp
from jax import lax
from jax.experimental import pallas as pl
from jax.experimental.pallas import tpu as pltpu
from jax.experimental.pallas import tpu_sc as plsc

NC = 2
NS = 16
L = 16
NW = NC * NS

CP = 16
TW = 16
P = 256
NG = P // L
ROWS = 8 * P
IDXW = 128
NCPY = ROWS // IDXW

JB = 8


def _build_body(d0, d1, d2, c_in, grid_hbm, tab_hbm,
                chan_a, chan_b, out_a, out_b, sem_ia, sem_ib, sem_oa, sem_ob):
    core = lax.axis_index("c")
    sub = lax.axis_index("s")
    wid = sub * NC + core
    islabs = d0 // NW
    njb = d1 // JB
    nchunk = islabs * njb
    m = nchunk // 2
    s_vox = JB * d2
    iota = lax.iota(jnp.int32, L)

    def fire_in(t, chan, sem):
        i = wid * islabs + t // njb
        jb = t % njb
        for c in range(c_in):
            pltpu.async_copy(grid_hbm.at[0, c, i, pl.ds(jb * JB, JB)],
                             chan.at[c], sem)

    def wait_in(chan, sem):
        for c in range(c_in):
            pltpu.make_async_copy(grid_hbm.at[0, c, 0, pl.ds(0, JB)],
                                  chan.at[c], sem).wait()

    colvs = []
    for d in range(c_in):
        cv = iota + d
        cv = jnp.where(cv >= c_in, cv - c_in, cv)
        cv = jnp.where(cv >= c_in, cv - c_in, cv)
        colvs.append(cv)
    voxvs = [iota + kg * L for kg in range(d2 // L)]

    def interleave(chan, outb):
        def jl_body(jl, carry):
            jsplat = jnp.full((L,), 0, jnp.int32) + jl
            for kg in range(d2 // L):
                rowv = voxvs[kg] + jl * d2
                for d in range(c_in):
                    val = plsc.load_gather(chan, [colvs[d], jsplat, voxvs[kg]])
                    plsc.store_scatter(outb, [rowv, colvs[d]], val)
            return carry
        lax.fori_loop(0, JB, jl_body, 0)

    def fire_out(t, outb, sem):
        i = wid * islabs + t // njb
        jb = t % njb
        vbase = (i * d1 + jb * JB) * d2
        pltpu.async_copy(outb, tab_hbm.at[pl.ds(vbase, s_vox)], sem)

    def wait_out(outb, sem):
        pltpu.make_async_copy(outb, tab_hbm.at[pl.ds(0, s_vox)], sem).wait()

    fire_in(0, chan_a, sem_ia)
    fire_in(1, chan_b, sem_ib)

    def pair_body(i2, carry):
        a = 2 * i2
        b = a + 1
        wait_in(chan_a, sem_ia)

        @pl.when(i2 > 0)
        def _():
            wait_out(out_a, sem_oa)

        interleave(chan_a, out_a)
        fire_out(a, out_a, sem_oa)

        @pl.when(i2 < m - 1)
        def _():
            fire_in(a + 2, chan_a, sem_ia)

        wait_in(chan_b, sem_ib)

        @pl.when(i2 > 0)
        def _():
            wait_out(out_b, sem_ob)

        interleave(chan_b, out_b)
        fire_out(b, out_b, sem_ob)

        @pl.when(i2 < m - 1)
        def _():
            fire_in(b + 2, chan_b, sem_ib)

        return carry

    lax.fori_loop(0, m, pair_body, 0)
    wait_out(out_a, sem_oa)
    wait_out(out_b, sem_ob)


def _sample_body(d0, d1, d2, c_out,
                 grid_hbm, xyz_hbm, coef_hbm, out_hbm,
                 coef_v, xyz_a, xyz_b, idx_a, idx_b, w_a, w_b,
                 vals_a, vals_b, out_a, out_b,
                 sem_xa, sem_xb, sem_ga, sem_gb, sem_oa, sem_ob):
    core = lax.axis_index("c")
    sub = lax.axis_index("s")
    wid = sub * NC + core
    n = out_hbm.shape[0]
    per_w = n // NW
    nchunk = per_w // P
    m = nchunk // 2
    wbase = wid * per_w

    pltpu.sync_copy(coef_hbm, coef_v)

    iota = lax.iota(jnp.int32, L)
    zero = jnp.full((L,), 0, jnp.int32)
    one = jnp.full((L,), 1, jnp.int32)
    two = jnp.full((L,), 2, jnp.int32)

    def fire_xyz(t, buf, sem):
        pltpu.async_copy(xyz_hbm.at[pl.ds(wbase + t * P, P)], buf, sem)

    def wait_xyz(buf, sem):
        pltpu.make_async_copy(xyz_hbm.at[pl.ds(0, P)], buf, sem).wait()

    def phase1(xyz_v, idx_v, w_v):
        def g_body(g, carry):
            s = g * L
            rowp = iota + s
            x = plsc.load_gather(xyz_v, [rowp, zero])
            y = plsc.load_gather(xyz_v, [rowp, one])
            z = plsc.load_gather(xyz_v, [rowp, two])
            px = x * coef_v[pl.ds(0, L)] + coef_v[pl.ds(3 * L, L)]
            py = y * coef_v[pl.ds(L, L)] + coef_v[pl.ds(4 * L, L)]
            pz = z * coef_v[pl.ds(2 * L, L)] + coef_v[pl.ds(5 * L, L)]
            valid = ((px >= 0.0) & (px <= d0 - 1.0)
                     & (py >= 0.0) & (py <= d1 - 1.0)
                     & (pz >= 0.0) & (pz <= d2 - 1.0))
            validf = jnp.where(valid, 1.0, 0.0).astype(jnp.float32)
            px = jnp.clip(px, 0.0, d0 - 1.0)
            py = jnp.clip(py, 0.0, d1 - 1.0)
            pz = jnp.clip(pz, 0.0, d2 - 1.0)
            ix = jnp.minimum(px.astype(jnp.int32), d0 - 2)
            iy = jnp.minimum(py.astype(jnp.int32), d1 - 2)
            iz = jnp.minimum(pz.astype(jnp.int32), d2 - 2)
            fx = px - ix.astype(jnp.float32)
            fy = py - iy.astype(jnp.float32)
            fz = pz - iz.astype(jnp.float32)
            gxv = (1.0 - fx) * validf
            fxv = fx * validf
            gy = 1.0 - fy
            gz = 1.0 - fz
            a_ = gy * gz
            b_ = gy * fz
            c_ = fy * gz
            d_ = fy * fz
            w8 = (gxv * a_, gxv * b_, gxv * c_, gxv * d_,
                  fxv * a_, fxv * b_, fxv * c_, fxv * d_)
            rbase = ix * (d1 * d2) + iy * d2 + iz
            offs = (0, 1, d2, d2 + 1,
                    d1 * d2, d1 * d2 + 1, d1 * d2 + d2, d1 * d2 + d2 + 1)
            for k in range(8):
                idx_v[pl.ds(k * P + s, L)] = rbase + offs[k]
                w_v[pl.ds(k * P + s, L)] = w8[k]
            return carry

        lax.fori_loop(0, NG, g_body, 0)

    def fire_g(idx_v, vals_v, sem):
        for j in range(NCPY):
            pltpu.async_copy(
                grid_hbm.at[idx_v.at[pl.ds(j * IDXW, IDXW)]],
                vals_v.at[pl.ds(j * IDXW, IDXW)],
                sem)

    def wait_g(idx_v, vals_v, sem):
        for j in range(NCPY):
            pltpu.make_async_copy(
                grid_hbm.at[idx_v.at[pl.ds(j * IDXW, IDXW)]],
                vals_v.at[pl.ds(j * IDXW, IDXW)],
                sem).wait()

    def phase3(w_v, vals_v, out_v):
        colvs = []
        for d in range(c_out):
            cv = iota + d
            cv = jnp.where(cv >= c_out, cv - c_out, cv)
            cv = jnp.where(cv >= c_out, cv - c_out, cv)
            colvs.append(cv)
        def g_body(g, carry):
            s = g * L
            rows = [iota + (k * P + s) for k in range(8)]
            wk = [w_v[pl.ds(k * P + s, L)] for k in range(8)]
            pidx = iota + s
            for d in range(c_out):
                acc = wk[0] * plsc.load_gather(vals_v, [rows[0], colvs[d]])
                for k in range(1, 8):
                    acc = acc + wk[k] * plsc.load_gather(
                        vals_v, [rows[k], colvs[d]])
                plsc.store_scatter(out_v, [pidx, colvs[d]], acc)
            return carry

        lax.fori_loop(0, NG, g_body, 0)

    def fire_out(t, out_v, sem):
        pltpu.async_copy(out_v, out_hbm.at[pl.ds(wbase + t * P, P)], sem)

    def wait_out(out_v, sem):
        pltpu.make_async_copy(out_v, out_hbm.at[pl.ds(0, P)], sem).wait()

    fire_xyz(0, xyz_a, sem_xa)
    wait_xyz(xyz_a, sem_xa)
    phase1(xyz_a, idx_a, w_a)
    fire_g(idx_a, vals_a, sem_ga)
    fire_xyz(1, xyz_b, sem_xb)

    def pair_body(i2, carry):
        a = 2 * i2
        b = a + 1
        wait_xyz(xyz_b, sem_xb)
        phase1(xyz_b, idx_b, w_b)
        fire_g(idx_b, vals_b, sem_gb)

        @pl.when(i2 < m - 1)
        def _():
            fire_xyz(a + 2, xyz_a, sem_xa)

        wait_g(idx_a, vals_a, sem_ga)

        @pl.when(i2 > 0)
        def _():
            wait_out(out_a, sem_oa)

        phase3(w_a, vals_a, out_a)
        fire_out(a, out_a, sem_oa)

        @pl.when(i2 < m - 1)
        def _():
            wait_xyz(xyz_a, sem_xa)
            phase1(xyz_a, idx_a, w_a)
            fire_g(idx_a, vals_a, sem_ga)
            fire_xyz(b + 2, xyz_b, sem_xb)

        wait_g(idx_b, vals_b, sem_gb)

        @pl.when(i2 > 0)
        def _():
            wait_out(out_b, sem_ob)

        phase3(w_b, vals_b, out_b)
        fire_out(b, out_b, sem_ob)
        return carry

    lax.fori_loop(0, m, pair_body, 0)
    wait_out(out_a, sem_oa)
    wait_out(out_b, sem_ob)


def kernel(xyz, grid, xyz_min, xyz_max):
    channels = grid.shape[1]
    shape = xyz.shape[:-1]
    pts = xyz.reshape(-1, 3)
    n = pts.shape[0]
    d0, d1, d2 = grid.shape[2:]
    v = d0 * d1 * d2
    assert n % (NW * 2 * P) == 0 and d0 % NW == 0 and d1 % JB == 0
    assert d2 % L == 0 and (d0 // NW) * (d1 // JB) % 2 == 0

    mesh = plsc.VectorSubcoreMesh(core_axis_name="c", subcore_axis_name="s",
                                  num_cores=NC, num_subcores=NS)
    sc_params = pltpu.CompilerParams(needs_layout_passes=False,
                                     use_tc_tiling_on_sc=False)

    build = pl.kernel(
        functools.partial(_build_body, d0, d1, d2, channels),
        out_type=jax.ShapeDtypeStruct((v, TW), jnp.float32),
        mesh=mesh,
        scratch_types=[
            pltpu.VMEM((channels, JB, d2), jnp.float32),
            pltpu.VMEM((channels, JB, d2), jnp.float32),
            pltpu.VMEM((JB * d2, TW), jnp.float32),
            pltpu.VMEM((JB * d2, TW), jnp.float32),
            pltpu.SemaphoreType.DMA,
            pltpu.SemaphoreType.DMA,
            pltpu.SemaphoreType.DMA,
            pltpu.SemaphoreType.DMA,
        ],
        compiler_params=sc_params,
    )
    grid_l = build(grid)

    sizes = jnp.array([d0 - 1, d1 - 1, d2 - 1], dtype=jnp.float32)
    scale = sizes / (xyz_max - xyz_min)
    off = -xyz_min * scale
    coef = jnp.concatenate([scale, off, jnp.zeros((2,), jnp.float32)])
    coef = jnp.broadcast_to(coef[:, None], (8, L)).astype(jnp.float32)
    coef = coef.reshape(8 * L)

    sample = pl.kernel(
        functools.partial(_sample_body, d0, d1, d2, channels),
        out_type=jax.ShapeDtypeStruct((n, channels), jnp.float32),
        mesh=mesh,
        scratch_types=[
            pltpu.VMEM((8 * L,), jnp.float32),
            pltpu.VMEM((P, 3), jnp.float32),
            pltpu.VMEM((P, 3), jnp.float32),
            pltpu.VMEM((ROWS,), jnp.int32),
            pltpu.VMEM((ROWS,), jnp.int32),
            pltpu.VMEM((ROWS,), jnp.float32),
            pltpu.VMEM((ROWS,), jnp.float32),
            pltpu.VMEM((ROWS, TW), jnp.float32),
            pltpu.VMEM((ROWS, TW), jnp.float32),
            pltpu.VMEM((P, channels), jnp.float32),
            pltpu.VMEM((P, channels), jnp.float32),
            pltpu.SemaphoreType.DMA,
            pltpu.SemaphoreType.DMA,
            pltpu.SemaphoreType.DMA,
            pltpu.SemaphoreType.DMA,
            pltpu.SemaphoreType.DMA,
            pltpu.SemaphoreType.DMA,
        ],
        compiler_params=sc_params,
    )
    out = sample(grid_l, pts, coef)
    out = out.reshape(*shape, channels)
    if channels == 1:
        out = out.squeeze(-1)
    return out


# speedup vs baseline: 1.0267x; 1.0267x over previous
import functools

import jax
import jax.numpy as jnp
from jax import lax
from jax.experimental import pallas as pl
from jax.experimental.pallas import tpu as pltpu
from jax.experimental.pallas import tpu_sc as plsc

NC = 2
NS = 16
L = 16
NW = NC * NS

CP = 16
TW = 16
P = 256
NG = P // L
ROWS = 8 * P
IDXW = 128
NCPY = ROWS // IDXW

JB = 8


def _build_body(d0, d1, d2, c_in, grid_hbm, tab_hbm,
                chan_a, chan_b, out_a, out_b, sem_ia, sem_ib, sem_oa, sem_ob):
    core = lax.axis_index("c")
    sub = lax.axis_index("s")
    wid = sub * NC + core
    islabs = d0 // NW
    njb = d1 // JB
    nchunk = islabs * njb
    m = nchunk // 2
    s_vox = JB * d2
    iota = lax.iota(jnp.int32, L)

    def fire_in(t, chan, sem):
        i = wid * islabs + t // njb
        jb = t % njb
        for c in range(c_in):
            pltpu.async_copy(grid_hbm.at[0, c, i, pl.ds(jb * JB, JB)],
                             chan.at[c], sem)

    def wait_in(chan, sem):
        for c in range(c_in):
            pltpu.make_async_copy(grid_hbm.at[0, c, 0, pl.ds(0, JB)],
                                  chan.at[c], sem).wait()

    def interleave(chan, outb):
        def jl_body(jl, carry):
            for kg in range(d2 // L):
                rowv = iota + (jl * d2 + kg * L)
                for c in range(c_in):
                    val = chan[c, jl, pl.ds(kg * L, L)]
                    plsc.store_scatter(
                        outb, [rowv, jnp.full((L,), c, jnp.int32)], val)
            return carry
        lax.fori_loop(0, JB, jl_body, 0)

    def fire_out(t, outb, sem):
        i = wid * islabs + t // njb
        jb = t % njb
        vbase = (i * d1 + jb * JB) * d2
        pltpu.async_copy(outb, tab_hbm.at[pl.ds(vbase, s_vox)], sem)

    def wait_out(outb, sem):
        pltpu.make_async_copy(outb, tab_hbm.at[pl.ds(0, s_vox)], sem).wait()

    fire_in(0, chan_a, sem_ia)
    fire_in(1, chan_b, sem_ib)

    def pair_body(i2, carry):
        a = 2 * i2
        b = a + 1
        wait_in(chan_a, sem_ia)

        @pl.when(i2 > 0)
        def _():
            wait_out(out_a, sem_oa)

        interleave(chan_a, out_a)
        fire_out(a, out_a, sem_oa)

        @pl.when(i2 < m - 1)
        def _():
            fire_in(a + 2, chan_a, sem_ia)

        wait_in(chan_b, sem_ib)

        @pl.when(i2 > 0)
        def _():
            wait_out(out_b, sem_ob)

        interleave(chan_b, out_b)
        fire_out(b, out_b, sem_ob)

        @pl.when(i2 < m - 1)
        def _():
            fire_in(b + 2, chan_b, sem_ib)

        return carry

    lax.fori_loop(0, m, pair_body, 0)
    wait_out(out_a, sem_oa)
    wait_out(out_b, sem_ob)


def _sample_body(d0, d1, d2, c_out,
                 grid_hbm, xyz_hbm, coef_hbm, out_hbm,
                 coef_v, xyz_a, xyz_b, idx_a, idx_b, w_a, w_b,
                 vals_a, vals_b, out_a, out_b,
                 sem_xa, sem_xb, sem_ga, sem_gb, sem_oa, sem_ob):
    core = lax.axis_index("c")
    sub = lax.axis_index("s")
    wid = sub * NC + core
    n = out_hbm.shape[0]
    per_w = n // NW
    nchunk = per_w // P
    m = nchunk // 2
    wbase = wid * per_w

    pltpu.sync_copy(coef_hbm, coef_v)

    iota = lax.iota(jnp.int32, L)
    zero = jnp.full((L,), 0, jnp.int32)
    one = jnp.full((L,), 1, jnp.int32)
    two = jnp.full((L,), 2, jnp.int32)

    def fire_xyz(t, buf, sem):
        pltpu.async_copy(xyz_hbm.at[pl.ds(wbase + t * P, P)], buf, sem)

    def wait_xyz(buf, sem):
        pltpu.make_async_copy(xyz_hbm.at[pl.ds(0, P)], buf, sem).wait()

    def phase1(xyz_v, idx_v, w_v):
        def g_body(g, carry):
            s = g * L
            rowp = iota + s
            x = plsc.load_gather(xyz_v, [rowp, zero])
            y = plsc.load_gather(xyz_v, [rowp, one])
            z = plsc.load_gather(xyz_v, [rowp, two])
            px = x * coef_v[pl.ds(0, L)] + coef_v[pl.ds(3 * L, L)]
            py = y * coef_v[pl.ds(L, L)] + coef_v[pl.ds(4 * L, L)]
            pz = z * coef_v[pl.ds(2 * L, L)] + coef_v[pl.ds(5 * L, L)]
            valid = ((px >= 0.0) & (px <= d0 - 1.0)
                     & (py >= 0.0) & (py <= d1 - 1.0)
                     & (pz >= 0.0) & (pz <= d2 - 1.0))
            validf = jnp.where(valid, 1.0, 0.0).astype(jnp.float32)
            px = jnp.clip(px, 0.0, d0 - 1.0)
            py = jnp.clip(py, 0.0, d1 - 1.0)
            pz = jnp.clip(pz, 0.0, d2 - 1.0)
            ix = jnp.minimum(px.astype(jnp.int32), d0 - 2)
            iy = jnp.minimum(py.astype(jnp.int32), d1 - 2)
            iz = jnp.minimum(pz.astype(jnp.int32), d2 - 2)
            fx = px - ix.astype(jnp.float32)
            fy = py - iy.astype(jnp.float32)
            fz = pz - iz.astype(jnp.float32)
            gxv = (1.0 - fx) * validf
            fxv = fx * validf
            gy = 1.0 - fy
            gz = 1.0 - fz
            a_ = gy * gz
            b_ = gy * fz
            c_ = fy * gz
            d_ = fy * fz
            w8 = (gxv * a_, gxv * b_, gxv * c_, gxv * d_,
                  fxv * a_, fxv * b_, fxv * c_, fxv * d_)
            rbase = ix * (d1 * d2) + iy * d2 + iz
            offs = (0, 1, d2, d2 + 1,
                    d1 * d2, d1 * d2 + 1, d1 * d2 + d2, d1 * d2 + d2 + 1)
            for k in range(8):
                idx_v[pl.ds(k * P + s, L)] = rbase + offs[k]
                w_v[pl.ds(k * P + s, L)] = w8[k]
            return carry

        lax.fori_loop(0, NG, g_body, 0)

    def fire_g(idx_v, vals_v, sem):
        for j in range(NCPY):
            pltpu.async_copy(
                grid_hbm.at[idx_v.at[pl.ds(j * IDXW, IDXW)]],
                vals_v.at[pl.ds(j * IDXW, IDXW)],
                sem)

    def wait_g(idx_v, vals_v, sem):
        for j in range(NCPY):
            pltpu.make_async_copy(
                grid_hbm.at[idx_v.at[pl.ds(j * IDXW, IDXW)]],
                vals_v.at[pl.ds(j * IDXW, IDXW)],
                sem).wait()

    def phase3(w_v, vals_v, out_v):
        colvs = []
        for d in range(c_out):
            cv = iota + d
            cv = jnp.where(cv >= c_out, cv - c_out, cv)
            cv = jnp.where(cv >= c_out, cv - c_out, cv)
            colvs.append(cv)
        def g_body(g, carry):
            s = g * L
            rows = [iota + (k * P + s) for k in range(8)]
            wk = [w_v[pl.ds(k * P + s, L)] for k in range(8)]
            pidx = iota + s
            for d in range(c_out):
                acc = wk[0] * plsc.load_gather(vals_v, [rows[0], colvs[d]])
                for k in range(1, 8):
                    acc = acc + wk[k] * plsc.load_gather(
                        vals_v, [rows[k], colvs[d]])
                plsc.store_scatter(out_v, [pidx, colvs[d]], acc)
            return carry

        lax.fori_loop(0, NG, g_body, 0)

    def fire_out(t, out_v, sem):
        pltpu.async_copy(out_v, out_hbm.at[pl.ds(wbase + t * P, P)], sem)

    def wait_out(out_v, sem):
        pltpu.make_async_copy(out_v, out_hbm.at[pl.ds(0, P)], sem).wait()

    fire_xyz(0, xyz_a, sem_xa)
    wait_xyz(xyz_a, sem_xa)
    phase1(xyz_a, idx_a, w_a)
    fire_g(idx_a, vals_a, sem_ga)
    fire_xyz(1, xyz_b, sem_xb)

    def pair_body(i2, carry):
        a = 2 * i2
        b = a + 1
        wait_xyz(xyz_b, sem_xb)
        phase1(xyz_b, idx_b, w_b)
        fire_g(idx_b, vals_b, sem_gb)

        @pl.when(i2 < m - 1)
        def _():
            fire_xyz(a + 2, xyz_a, sem_xa)

        wait_g(idx_a, vals_a, sem_ga)

        @pl.when(i2 > 0)
        def _():
            wait_out(out_a, sem_oa)

        phase3(w_a, vals_a, out_a)
        fire_out(a, out_a, sem_oa)

        @pl.when(i2 < m - 1)
        def _():
            wait_xyz(xyz_a, sem_xa)
            phase1(xyz_a, idx_a, w_a)
            fire_g(idx_a, vals_a, sem_ga)
            fire_xyz(b + 2, xyz_b, sem_xb)

        wait_g(idx_b, vals_b, sem_gb)

        @pl.when(i2 > 0)
        def _():
            wait_out(out_b, sem_ob)

        phase3(w_b, vals_b, out_b)
        fire_out(b, out_b, sem_ob)
        return carry

    lax.fori_loop(0, m, pair_body, 0)
    wait_out(out_a, sem_oa)
    wait_out(out_b, sem_ob)


def kernel(xyz, grid, xyz_min, xyz_max):
    channels = grid.shape[1]
    shape = xyz.shape[:-1]
    pts = xyz.reshape(-1, 3)
    n = pts.shape[0]
    d0, d1, d2 = grid.shape[2:]
    v = d0 * d1 * d2
    assert n % (NW * 2 * P) == 0 and d0 % NW == 0 and d1 % JB == 0
    assert d2 % L == 0 and (d0 // NW) * (d1 // JB) % 2 == 0

    mesh = plsc.VectorSubcoreMesh(core_axis_name="c", subcore_axis_name="s",
                                  num_cores=NC, num_subcores=NS)
    sc_params = pltpu.CompilerParams(needs_layout_passes=False,
                                     use_tc_tiling_on_sc=False)

    build = pl.kernel(
        functools.partial(_build_body, d0, d1, d2, channels),
        out_type=jax.ShapeDtypeStruct((v, TW), jnp.float32),
        mesh=mesh,
        scratch_types=[
            pltpu.VMEM((channels, JB, d2), jnp.float32),
            pltpu.VMEM((channels, JB, d2), jnp.float32),
            pltpu.VMEM((JB * d2, TW), jnp.float32),
            pltpu.VMEM((JB * d2, TW), jnp.float32),
            pltpu.SemaphoreType.DMA,
            pltpu.SemaphoreType.DMA,
            pltpu.SemaphoreType.DMA,
            pltpu.SemaphoreType.DMA,
        ],
        compiler_params=sc_params,
    )
    grid_l = build(grid)

    sizes = jnp.array([d0 - 1, d1 - 1, d2 - 1], dtype=jnp.float32)
    scale = sizes / (xyz_max - xyz_min)
    off = -xyz_min * scale
    coef = jnp.concatenate([scale, off, jnp.zeros((2,), jnp.float32)])
    coef = jnp.broadcast_to(coef[:, None], (8, L)).astype(jnp.float32)
    coef = coef.reshape(8 * L)

    sample = pl.kernel(
        functools.partial(_sample_body, d0, d1, d2, channels),
        out_type=jax.ShapeDtypeStruct((n, channels), jnp.float32),
        mesh=mesh,
        scratch_types=[
            pltpu.VMEM((8 * L,), jnp.float32),
            pltpu.VMEM((P, 3), jnp.float32),
            pltpu.VMEM((P, 3), jnp.float32),
            pltpu.VMEM((ROWS,), jnp.int32),
            pltpu.VMEM((ROWS,), jnp.int32),
            pltpu.VMEM((ROWS,), jnp.float32),
            pltpu.VMEM((ROWS,), jnp.float32),
            pltpu.VMEM((ROWS, TW), jnp.float32),
            pltpu.VMEM((ROWS, TW), jnp.float32),
            pltpu.VMEM((P, channels), jnp.float32),
            pltpu.VMEM((P, channels), jnp.float32),
            pltpu.SemaphoreType.DMA,
            pltpu.SemaphoreType.DMA,
            pltpu.SemaphoreType.DMA,
            pltpu.SemaphoreType.DMA,
            pltpu.SemaphoreType.DMA,
            pltpu.SemaphoreType.DMA,
        ],
        compiler_params=sc_params,
    )
    out = sample(grid_l, pts, coef)
    out = out.reshape(*shape, channels)
    if channels == 1:
        out = out.squeeze(-1)
    return out
